# baseline (speedup 1.0000x reference)
import functools

import jax
import jax.numpy as jnp
from jax.experimental import pallas as pl
from jax.experimental.pallas import tpu as pltpu

BSZ = 4096
BN = 512
GRID = BSZ // BN
NEG_INF = float("-inf")


def _nce_kernel(ctx_ref, q_ref, out_ref, rmax_ref, rsum_ref, clse_ref, diag_ref):
    j = pl.program_id(0)

    @pl.when(j == 0)
    def _init():
        rmax_ref[...] = jnp.full((BSZ, 1), NEG_INF, jnp.float32)
        rsum_ref[...] = jnp.zeros((BSZ, 1), jnp.float32)

    tile = jax.lax.dot_general(
        ctx_ref[...].astype(jnp.bfloat16), q_ref[...].astype(jnp.bfloat16),
        dimension_numbers=(((1,), (1,)), ((), ())),
        preferred_element_type=jnp.float32,
    )

    cmax = jnp.max(tile, axis=0, keepdims=True)
    csum = jnp.sum(jnp.exp(tile - cmax), axis=0, keepdims=True)
    clse_ref[:, pl.ds(j * BN, BN)] = cmax + jnp.log(csum)

    rows = jax.lax.broadcasted_iota(jnp.int32, (BSZ, BN), 0)
    cols = jax.lax.broadcasted_iota(jnp.int32, (BSZ, BN), 1)
    mask = rows == cols + j * BN
    diag_ref[:, pl.ds(j * BN, BN)] = jnp.sum(
        jnp.where(mask, tile, 0.0), axis=0, keepdims=True)

    tmax = jnp.max(tile, axis=1, keepdims=True)
    new_max = jnp.maximum(rmax_ref[...], tmax)
    rsum_ref[...] = (rsum_ref[...] * jnp.exp(rmax_ref[...] - new_max)
                     + jnp.sum(jnp.exp(tile - new_max), axis=1, keepdims=True))
    rmax_ref[...] = new_max

    @pl.when(j == GRID - 1)
    def _finish():
        row_lse = rmax_ref[...] + jnp.log(rsum_ref[...])
        row_lse_t = jax.lax.dot_general(
            jnp.ones((1, 1), jnp.float32), row_lse,
            dimension_numbers=(((1,), (1,)), ((), ())),
            preferred_element_type=jnp.float32,
        )
        denom = jnp.logaddexp(row_lse_t, clse_ref[...])
        dsum = jnp.sum(denom, axis=1, keepdims=True)
        nsum = jnp.sum(diag_ref[...], axis=1, keepdims=True)
        out_ref[...] = (dsum - nsum) / BSZ


@jax.jit
def kernel(contexts, queries):
    out = pl.pallas_call(
        _nce_kernel,
        grid=(GRID,),
        in_specs=[
            pl.BlockSpec((BSZ, 1024), lambda j: (0, 0)),
            pl.BlockSpec((BN, 1024), lambda j: (j, 0)),
        ],
        out_specs=pl.BlockSpec((1, 1), lambda j: (0, 0)),
        out_shape=jax.ShapeDtypeStruct((1, 1), jnp.float32),
        scratch_shapes=[
            pltpu.VMEM((BSZ, 1), jnp.float32),
            pltpu.VMEM((BSZ, 1), jnp.float32),
            pltpu.VMEM((1, BSZ), jnp.float32),
            pltpu.VMEM((1, BSZ), jnp.float32),
        ],
    )(contexts, queries)
    return out[0, 0]


# speedup vs baseline: 1.7549x; 1.7549x over previous
import jax
import jax.numpy as jnp
from jax.experimental import pallas as pl
from jax.experimental.pallas import tpu as pltpu

BSZ = 4096
BN = 512
GRID = BSZ // BN
NEG_INF = float("-inf")


def _nce_kernel(ctx_ref, q_ref, out_ref,
                ctx_bf16_ref, rmax_ref, rsum_ref, clse_ref, diag_ref):
    j = pl.program_id(0)

    @pl.when(j == 0)
    def _init():
        ctx_bf16_ref[...] = ctx_ref[...].astype(jnp.bfloat16)
        rmax_ref[...] = jnp.full((BSZ, 1), NEG_INF, jnp.float32)
        rsum_ref[...] = jnp.zeros((BSZ, 1), jnp.float32)

    tile = jax.lax.dot_general(
        ctx_bf16_ref[...], q_ref[...].astype(jnp.bfloat16),
        dimension_numbers=(((1,), (1,)), ((), ())),
        preferred_element_type=jnp.float32,
    )

    cmax = jnp.max(tile, axis=0, keepdims=True)
    mg = jnp.max(cmax)
    e = jnp.exp(tile - mg)

    csum = jnp.sum(e, axis=0, keepdims=True)
    clse_ref[:, pl.ds(j * BN, BN)] = mg + jnp.log(csum)

    diag_ref[pl.ds(j * BN, BN), :] = jnp.sum(
        ctx_ref[pl.ds(j * BN, BN), :] * q_ref[...], axis=1, keepdims=True)

    rpart = jnp.sum(e, axis=1, keepdims=True)
    new_max = jnp.maximum(rmax_ref[...], mg)
    rsum_ref[...] = (rsum_ref[...] * jnp.exp(rmax_ref[...] - new_max)
                     + rpart * jnp.exp(mg - new_max))
    rmax_ref[...] = new_max

    @pl.when(j == GRID - 1)
    def _finish():
        row_lse = rmax_ref[...] + jnp.log(rsum_ref[...])
        row_lse_t = jax.lax.dot_general(
            jnp.ones((1, 1), jnp.float32), row_lse,
            dimension_numbers=(((1,), (1,)), ((), ())),
            preferred_element_type=jnp.float32,
        )
        denom = jnp.logaddexp(row_lse_t, clse_ref[...])
        dsum = jnp.sum(denom, axis=1, keepdims=True)
        nsum = jnp.sum(diag_ref[...], axis=0, keepdims=True)
        out_ref[...] = (dsum - nsum) / BSZ


@jax.jit
def kernel(contexts, queries):
    out = pl.pallas_call(
        _nce_kernel,
        grid=(GRID,),
        in_specs=[
            pl.BlockSpec((BSZ, 1024), lambda j: (0, 0)),
            pl.BlockSpec((BN, 1024), lambda j: (j, 0)),
        ],
        out_specs=pl.BlockSpec((1, 1), lambda j: (0, 0)),
        out_shape=jax.ShapeDtypeStruct((1, 1), jnp.float32),
        scratch_shapes=[
            pltpu.VMEM((BSZ, 1024), jnp.bfloat16),
            pltpu.VMEM((BSZ, 1), jnp.float32),
            pltpu.VMEM((BSZ, 1), jnp.float32),
            pltpu.VMEM((1, BSZ), jnp.float32),
            pltpu.VMEM((BSZ, 1), jnp.float32),
        ],
    )(contexts, queries)
    return out[0, 0]
